# baseline (speedup 1.0000x reference)
import jax
import jax.numpy as jnp
from jax import lax
from jax.experimental import pallas as pl
from jax.experimental.pallas import tpu as pltpu
from jax.experimental.pallas import tpu_sc as plsc

MAX_LEN = 8192
BS = 2048

B = 4
S = 8192
D = 1024
NW = 32
RPW = S // NW
SUB = 16
NSUB = RPW // SUB


def _tc_posemb_kernel(x_ref, emb_ref, bnd_ref, out_ref):
    em = emb_ref[...]
    rolled = jnp.concatenate([em[1:], bnd_ref[0]], axis=0)
    out_ref[0] = x_ref[0] + rolled


def _tc_kernel(x, emb):
    B_, S_, D_ = x.shape
    nj = S_ // BS
    bnd = jnp.concatenate([emb[BS:S_:BS], emb[0:1]], axis=0).reshape(nj, 1, D_)
    return pl.pallas_call(
        _tc_posemb_kernel,
        grid=(nj, B_),
        in_specs=[
            pl.BlockSpec((1, BS, D_), lambda j, b: (b, j, 0)),
            pl.BlockSpec((BS, D_), lambda j, b: (j, 0)),
            pl.BlockSpec((1, 1, D_), lambda j, b: (j, 0, 0)),
        ],
        out_specs=pl.BlockSpec((1, BS, D_), lambda j, b: (b, j, 0)),
        out_shape=jax.ShapeDtypeStruct(x.shape, x.dtype),
        compiler_params=pltpu.CompilerParams(
            dimension_semantics=("arbitrary", "arbitrary"),
        ),
    )(x, emb, bnd)


def _sc_body(x_hbm, emb_hbm, out_hbm, ebuf, xbuf, e_sem, xin_sem, xout_sem):
    c = lax.axis_index("c")
    sid = lax.axis_index("s")
    wid = sid * 2 + c
    base = wid * RPW
    NI = NSUB * B

    def e_start(kk):
        s0 = base + kk * SUB
        slot = kk % 2

        @pl.when(s0 + SUB != S)
        def _():
            pltpu.make_async_copy(
                emb_hbm.at[pl.ds(s0, SUB + 8), :], ebuf.at[slot], e_sem.at[slot]
            ).start()

        @pl.when(s0 + SUB == S)
        def _():
            pltpu.make_async_copy(
                emb_hbm.at[pl.ds(s0, SUB), :],
                ebuf.at[slot, pl.ds(0, SUB), :],
                e_sem.at[slot],
            ).start()

    def e_wait(kk):
        s0 = base + kk * SUB
        slot = kk % 2

        @pl.when(s0 + SUB != S)
        def _():
            pltpu.make_async_copy(
                emb_hbm.at[pl.ds(s0, SUB + 8), :], ebuf.at[slot], e_sem.at[slot]
            ).wait()

        @pl.when(s0 + SUB == S)
        def _():
            pltpu.make_async_copy(
                emb_hbm.at[pl.ds(s0, SUB), :],
                ebuf.at[slot, pl.ds(0, SUB), :],
                e_sem.at[slot],
            ).wait()
            pltpu.sync_copy(emb_hbm.at[pl.ds(0, 1), :], ebuf.at[slot, pl.ds(SUB, 1), :])

    def x_in_start(ii):
        kk, bb, slot = ii >> 2, ii & 3, ii % 3
        pltpu.make_async_copy(
            x_hbm.at[bb, pl.ds(base + kk * SUB, SUB), :],
            xbuf.at[slot],
            xin_sem.at[slot],
        ).start()

    def x_out_desc(ii):
        kk, bb, slot = ii >> 2, ii & 3, ii % 3
        return pltpu.make_async_copy(
            xbuf.at[slot],
            out_hbm.at[bb, pl.ds(base + kk * SUB, SUB), :],
            xout_sem.at[slot],
        )

    e_start(0)
    x_in_start(0)

    def step(i, carry):
        kk, bb, slot = i >> 2, i & 3, i % 3

        @pl.when(i + 1 < NI)
        def _():
            @pl.when(i >= 2)
            def _():
                x_out_desc(i - 2).wait()

            x_in_start(i + 1)

            @pl.when((i + 1) & 3 == 0)
            def _():
                e_start((i + 1) >> 2)

        @pl.when(bb == 0)
        def _():
            e_wait(kk)

        pltpu.make_async_copy(
            x_hbm.at[bb, pl.ds(base + kk * SUB, SUB), :],
            xbuf.at[slot],
            xin_sem.at[slot],
        ).wait()

        eslot = kk % 2

        def row_loop(r, acc):
            def inner(cc, a2):
                ev = ebuf[eslot, r + 1, pl.ds(cc * 16, 16)]
                plsc.addupdate(xbuf.at[slot, r, pl.ds(cc * 16, 16)], ev)
                return a2
            lax.fori_loop(0, D // 16, inner, 0, unroll=16)
            return acc

        lax.fori_loop(0, SUB, row_loop, 0)
        x_out_desc(i).start()
        return carry

    lax.fori_loop(0, NI, step, 0)
    x_out_desc(NI - 3).wait()
    x_out_desc(NI - 2).wait()
    x_out_desc(NI - 1).wait()


def kernel(x, emb):
    return pl.kernel(
        _sc_body,
        out_type=jax.ShapeDtypeStruct(x.shape, x.dtype),
        mesh=plsc.VectorSubcoreMesh(core_axis_name="c", subcore_axis_name="s"),
        scratch_types=[
            pltpu.VMEM((2, SUB + 8, D), jnp.float32),
            pltpu.VMEM((3, SUB, D), jnp.float32),
            pltpu.SemaphoreType.DMA((2,)),
            pltpu.SemaphoreType.DMA((3,)),
            pltpu.SemaphoreType.DMA((3,)),
        ],
    )(x, emb)


# speedup vs baseline: 1.1411x; 1.1411x over previous
import jax
import jax.numpy as jnp
from jax import lax
from jax.experimental import pallas as pl
from jax.experimental.pallas import tpu as pltpu
from jax.experimental.pallas import tpu_sc as plsc

MAX_LEN = 8192
BS = 2048

B = 4
S = 8192
D = 1024
NW = 32
RPW = S // NW
SUB = 16
NSUB = RPW // SUB


def _tc_posemb_kernel(x_ref, emb_ref, bnd_ref, out_ref):
    em = emb_ref[...]
    rolled = jnp.concatenate([em[1:], bnd_ref[0]], axis=0)
    out_ref[0] = x_ref[0] + rolled


def _tc_kernel(x, emb, nb):
    nj = S // BS
    bnd = jnp.concatenate([emb[BS:S:BS], emb[0:1]], axis=0).reshape(nj, 1, D)
    return pl.pallas_call(
        _tc_posemb_kernel,
        grid=(nj, nb),
        in_specs=[
            pl.BlockSpec((1, BS, D), lambda j, b: (b, j, 0)),
            pl.BlockSpec((BS, D), lambda j, b: (j, 0)),
            pl.BlockSpec((1, 1, D), lambda j, b: (j, 0, 0)),
        ],
        out_specs=pl.BlockSpec((1, BS, D), lambda j, b: (b, j, 0)),
        out_shape=jax.ShapeDtypeStruct((nb, S, D), x.dtype),
        compiler_params=pltpu.CompilerParams(
            dimension_semantics=("arbitrary", "arbitrary"),
        ),
    )(x, emb, bnd)


def _make_sc_body(bat_ids):
    NB = len(bat_ids)

    def _sc_body(x_hbm, emb_hbm, out_hbm, ebuf, xbuf, e_sem, xin_sem, xout_sem):
        c = lax.axis_index("c")
        sid = lax.axis_index("s")
        wid = sid * 2 + c
        base = wid * RPW
        NI = NSUB * NB

        def split(ii):
            if NB == 1:
                return ii, bat_ids[0], 0, ii % 3
            return ii // NB, ii % NB, ii % NB, ii % 3

        def e_start(kk):
            s0 = base + kk * SUB
            slot = kk % 2

            @pl.when(s0 + SUB != S)
            def _():
                pltpu.make_async_copy(
                    emb_hbm.at[pl.ds(s0, SUB + 8), :], ebuf.at[slot], e_sem.at[slot]
                ).start()

            @pl.when(s0 + SUB == S)
            def _():
                pltpu.make_async_copy(
                    emb_hbm.at[pl.ds(s0, SUB), :],
                    ebuf.at[slot, pl.ds(0, SUB), :],
                    e_sem.at[slot],
                ).start()

        def e_wait(kk):
            s0 = base + kk * SUB
            slot = kk % 2

            @pl.when(s0 + SUB != S)
            def _():
                pltpu.make_async_copy(
                    emb_hbm.at[pl.ds(s0, SUB + 8), :], ebuf.at[slot], e_sem.at[slot]
                ).wait()

            @pl.when(s0 + SUB == S)
            def _():
                pltpu.make_async_copy(
                    emb_hbm.at[pl.ds(s0, SUB), :],
                    ebuf.at[slot, pl.ds(0, SUB), :],
                    e_sem.at[slot],
                ).wait()
                pltpu.sync_copy(
                    emb_hbm.at[pl.ds(0, 1), :], ebuf.at[slot, pl.ds(SUB, 1), :]
                )

        def x_in_start(ii):
            kk, xb, _, slot = split(ii)
            pltpu.make_async_copy(
                x_hbm.at[xb, pl.ds(base + kk * SUB, SUB), :],
                xbuf.at[slot],
                xin_sem.at[slot],
            ).start()

        def x_out_desc(ii):
            kk, _, ob, slot = split(ii)
            return pltpu.make_async_copy(
                xbuf.at[slot],
                out_hbm.at[ob, pl.ds(base + kk * SUB, SUB), :],
                xout_sem.at[slot],
            )

        e_start(0)
        x_in_start(0)

        def step(i, carry):
            kk, xb, _, slot = split(i)

            @pl.when(i + 1 < NI)
            def _():
                @pl.when(i >= 2)
                def _():
                    x_out_desc(i - 2).wait()

                x_in_start(i + 1)

                @pl.when((i + 1) % NB == 0)
                def _():
                    e_start((i + 1) // NB)

            @pl.when(i % NB == 0)
            def _():
                e_wait(kk)

            pltpu.make_async_copy(
                x_hbm.at[xb, pl.ds(base + kk * SUB, SUB), :],
                xbuf.at[slot],
                xin_sem.at[slot],
            ).wait()

            eslot = kk % 2

            def row_loop(r, acc):
                def inner(cc, a2):
                    ev = ebuf[eslot, r + 1, pl.ds(cc * 16, 16)]
                    plsc.addupdate(xbuf.at[slot, r, pl.ds(cc * 16, 16)], ev)
                    return a2
                lax.fori_loop(0, D // 16, inner, 0, unroll=8)
                return acc

            lax.fori_loop(0, SUB, row_loop, 0)
            x_out_desc(i).start()
            return carry

        lax.fori_loop(0, NI, step, 0)
        x_out_desc(NI - 3).wait()
        x_out_desc(NI - 2).wait()
        x_out_desc(NI - 1).wait()

    return _sc_body


def _sc_kernel(x, emb, bat_ids):
    return pl.kernel(
        _make_sc_body(bat_ids),
        out_type=jax.ShapeDtypeStruct((len(bat_ids), S, D), x.dtype),
        mesh=plsc.VectorSubcoreMesh(core_axis_name="c", subcore_axis_name="s"),
        scratch_types=[
            pltpu.VMEM((2, SUB + 8, D), jnp.float32),
            pltpu.VMEM((3, SUB, D), jnp.float32),
            pltpu.SemaphoreType.DMA((2,)),
            pltpu.SemaphoreType.DMA((3,)),
            pltpu.SemaphoreType.DMA((3,)),
        ],
    )(x, emb)


def kernel(x, emb):
    tc_out = _tc_kernel(x, emb, 3)
    sc_out = _sc_kernel(x, emb, (3,))
    return jnp.concatenate([tc_out, sc_out], axis=0)


# speedup vs baseline: 2.4483x; 2.1455x over previous
import jax
import jax.numpy as jnp
from jax.experimental import pallas as pl
from jax.experimental.pallas import tpu as pltpu

MAX_LEN = 8192
BS = 2048


def _posemb_kernel(x_ref, emb_hbm, bnd_ref, out_ref, ebuf, esem):
    j = pl.program_id(0)
    b = pl.program_id(1)
    nj = pl.num_programs(0)

    def emb_copy(jj, slot):
        return pltpu.make_async_copy(
            emb_hbm.at[pl.ds(jj * BS, BS), :], ebuf.at[slot], esem.at[slot]
        )

    @pl.when((j == 0) & (b == 0))
    def _():
        emb_copy(0, 0).start()

    @pl.when((b == 0) & (j + 1 < nj))
    def _():
        emb_copy(j + 1, (j + 1) % 2).start()

    @pl.when(b == 0)
    def _():
        emb_copy(j, j % 2).wait()

    em = ebuf[j % 2]
    rolled = jnp.concatenate([em[1:], bnd_ref[0]], axis=0)
    out_ref[0] = x_ref[0] + rolled


def kernel(x, emb):
    B, S, D = x.shape
    nj = S // BS
    bnd = jnp.concatenate([emb[BS:S:BS], emb[0:1]], axis=0).reshape(nj, 1, D)
    return pl.pallas_call(
        _posemb_kernel,
        grid=(nj, B),
        in_specs=[
            pl.BlockSpec((1, BS, D), lambda j, b: (b, j, 0)),
            pl.BlockSpec(memory_space=pl.MemorySpace.ANY),
            pl.BlockSpec((1, 1, D), lambda j, b: (j, 0, 0)),
        ],
        out_specs=pl.BlockSpec((1, BS, D), lambda j, b: (b, j, 0)),
        out_shape=jax.ShapeDtypeStruct(x.shape, x.dtype),
        scratch_shapes=[
            pltpu.VMEM((2, BS, D), jnp.float32),
            pltpu.SemaphoreType.DMA((2,)),
        ],
        compiler_params=pltpu.CompilerParams(
            dimension_semantics=("arbitrary", "arbitrary"),
        ),
    )(x, emb, bnd)


# speedup vs baseline: 2.4483x; 1.0000x over previous
import jax
import jax.numpy as jnp
from jax.experimental import pallas as pl
from jax.experimental.pallas import tpu as pltpu

MAX_LEN = 8192
BS = 2048


def _posemb_kernel(x_ref, emb_ref, bnd_ref, out_ref):
    em = emb_ref[...]
    rolled = jnp.concatenate([em[1:], bnd_ref[0]], axis=0)
    out_ref[0] = x_ref[0] + rolled


def kernel(x, emb):
    B, S, D = x.shape
    nj = S // BS
    bnd = jnp.concatenate([emb[BS:S:BS], emb[0:1]], axis=0).reshape(nj, 1, D)
    return pl.pallas_call(
        _posemb_kernel,
        grid=(nj, B),
        in_specs=[
            pl.BlockSpec((1, BS, D), lambda j, b: (b, j, 0)),
            pl.BlockSpec((BS, D), lambda j, b: (j, 0)),
            pl.BlockSpec((1, 1, D), lambda j, b: (j, 0, 0)),
        ],
        out_specs=pl.BlockSpec((1, BS, D), lambda j, b: (b, j, 0)),
        out_shape=jax.ShapeDtypeStruct(x.shape, x.dtype),
        compiler_params=pltpu.CompilerParams(
            dimension_semantics=("arbitrary", "arbitrary"),
        ),
    )(x, emb, bnd)
